# baseline (speedup 1.0000x reference)
import functools

import jax
import jax.numpy as jnp
from jax import lax
from jax.experimental import pallas as pl
from jax.experimental.pallas import tpu as pltpu
from jax.experimental.pallas import tpu_sc as plsc

_NW = 32
_T = 128
_LANES = 16


def kernel(x, E_relative_position):
    batch, seq, _ = x.shape
    attrs, edim = E_relative_position.shape
    rows = batch * seq
    rows_per_w = rows // _NW

    mesh = plsc.VectorSubcoreMesh(core_axis_name="c", subcore_axis_name="s")

    @functools.partial(
        pl.kernel,
        out_type=jax.ShapeDtypeStruct((rows, edim), jnp.float32),
        mesh=mesh,
        scratch_types=[
            pltpu.VMEM((_T, edim), jnp.float32),
            pltpu.SemaphoreType.DMA,
        ],
    )
    def sc_broadcast(table_hbm, out_hbm, buf, sem):
        wid = lax.axis_index("c") * 16 + lax.axis_index("s")
        base = wid * rows_per_w
        pltpu.async_copy(table_hbm, buf.at[pl.ds(0, attrs)], sem).wait()
        for r in range(attrs):
            for c in range(edim // _LANES):
                v = buf[r, pl.ds(c * _LANES, _LANES)]
                for k in range(1, _T // attrs):
                    buf[k * attrs + r, pl.ds(c * _LANES, _LANES)] = v
        writes = [
            pltpu.async_copy(
                buf, out_hbm.at[pl.ds(base + j * _T, _T)], sem
            )
            for j in range(rows_per_w // _T)
        ]
        for w in writes:
            w.wait()

    out = sc_broadcast(E_relative_position)
    return out.reshape(batch, seq, edim)


# speedup vs baseline: 4.6384x; 4.6384x over previous
import jax
import jax.numpy as jnp
from jax.experimental import pallas as pl
from jax.experimental.pallas import tpu as pltpu

_CHUNK = 64


def kernel(x, E_relative_position):
    batch, seq, _ = x.shape
    attrs, edim = E_relative_position.shape
    reps = batch * seq // attrs
    n_dma = reps // _CHUNK

    def body(tab_ref, out_hbm, buf, sem):
        buf[...] = jnp.broadcast_to(tab_ref[...][None], (_CHUNK, attrs, edim))
        copies = [
            pltpu.make_async_copy(
                buf, out_hbm.at[pl.ds(k * _CHUNK, _CHUNK)], sem
            )
            for k in range(n_dma)
        ]
        for c in copies:
            c.start()
        for c in copies:
            c.wait()

    out = pl.pallas_call(
        body,
        in_specs=[pl.BlockSpec(memory_space=pltpu.MemorySpace.VMEM)],
        out_specs=pl.BlockSpec(memory_space=pltpu.MemorySpace.HBM),
        out_shape=jax.ShapeDtypeStruct((reps, attrs, edim), jnp.float32),
        scratch_shapes=[
            pltpu.VMEM((_CHUNK, attrs, edim), jnp.float32),
            pltpu.SemaphoreType.DMA,
        ],
    )(E_relative_position)
    return out.reshape(batch, seq, edim)


# speedup vs baseline: 4.8272x; 1.0407x over previous
import jax
import jax.numpy as jnp
from jax.experimental import pallas as pl
from jax.experimental.pallas import tpu as pltpu

_CHUNK = 128


def kernel(x, E_relative_position):
    batch, seq, _ = x.shape
    attrs, edim = E_relative_position.shape
    reps = batch * seq // attrs
    n_dma = reps // _CHUNK

    def body(tab_ref, out_hbm, buf, sem):
        buf[...] = jnp.broadcast_to(tab_ref[...][None], (_CHUNK, attrs, edim))
        copies = [
            pltpu.make_async_copy(
                buf, out_hbm.at[pl.ds(k * _CHUNK, _CHUNK)], sem
            )
            for k in range(n_dma)
        ]
        for c in copies:
            c.start()
        for c in copies:
            c.wait()

    out = pl.pallas_call(
        body,
        in_specs=[pl.BlockSpec(memory_space=pltpu.MemorySpace.VMEM)],
        out_specs=pl.BlockSpec(memory_space=pltpu.MemorySpace.HBM),
        out_shape=jax.ShapeDtypeStruct((reps, attrs, edim), jnp.float32),
        scratch_shapes=[
            pltpu.VMEM((_CHUNK, attrs, edim), jnp.float32),
            pltpu.SemaphoreType.DMA,
        ],
    )(E_relative_position)
    return out.reshape(batch, seq, edim)
